# baseline (speedup 1.0000x reference)
import functools

import jax
import jax.numpy as jnp
from jax import lax
from jax.experimental import pallas as pl
from jax.experimental.pallas import tpu as pltpu
from jax.experimental.pallas import tpu_sc as plsc

N0, N1, N2 = 40960, 10240, 1024
D_IN, D_HID, D_OUT = 256, 32, 256
E1, E2 = 163840, 16384
NC, NS = 2, 16
NW = NC * NS
DA = 128
ONES_HI = D_HID + 16
CHUNK = 128


def _sc_mesh():
    return plsc.VectorSubcoreMesh(core_axis_name="c", subcore_axis_name="s")


def _gather_rows(table, idx3d, n_rows, d):
    per_w = n_rows // NW
    n_sub = per_w // 64

    assert n_sub % 2 == 1

    @functools.partial(
        pl.kernel,
        mesh=_sc_mesh(),
        out_type=jax.ShapeDtypeStruct((n_rows, d), jnp.float32),
        scratch_types=[
            pltpu.VMEM((n_sub, 64), jnp.int32),
            pltpu.VMEM((64, d), jnp.float32),
            pltpu.VMEM((64, d), jnp.float32),
            pltpu.SemaphoreType.DMA,
            pltpu.SemaphoreType.DMA,
        ],
    )
    def k(table_hbm, idx_hbm, out_hbm, idx_v, rows0, rows1, sem0, sem1):
        cid = lax.axis_index("c")
        sid = lax.axis_index("s")
        wid = sid * NC + cid
        base = wid * per_w
        pltpu.sync_copy(idx_hbm.at[wid], idx_v)
        last = n_sub - 1
        pltpu.async_copy(table_hbm.at[idx_v.at[0]], rows0, sem0)

        def body(jj, _):
            j1 = jj * 2 + 1
            j0n = jnp.minimum(j1 + 1, last)
            pltpu.async_copy(table_hbm.at[idx_v.at[j1]], rows1, sem1)
            pltpu.make_async_copy(
                table_hbm.at[idx_v.at[j1 - 1]], rows0, sem0).wait()
            pltpu.sync_copy(rows0, out_hbm.at[pl.ds(base + (j1 - 1) * 64, 64)])
            pltpu.async_copy(table_hbm.at[idx_v.at[j0n]], rows0, sem0)
            pltpu.make_async_copy(
                table_hbm.at[idx_v.at[j1]], rows1, sem1).wait()
            pltpu.sync_copy(rows1, out_hbm.at[pl.ds(base + j1 * 64, 64)])
            return 0

        lax.fori_loop(0, n_sub // 2, body, 0)
        pltpu.make_async_copy(table_hbm.at[idx_v.at[last]], rows0, sem0).wait()
        pltpu.sync_copy(rows0, out_hbm.at[pl.ds(base + last * 64, 64)])

    return k(table, idx3d)


def _seg_accum(zaug, src3d, dst3d, redirect):
    n_acc = (N2 + 64) if redirect else N2
    ch_per_w = src3d.shape[1]
    zrows = N2 // NS
    assert ch_per_w % 4 == 0
    NBUF = 4

    @functools.partial(
        pl.kernel,
        mesh=_sc_mesh(),
        out_type=jax.ShapeDtypeStruct((NC, N2, DA), jnp.float32),
        scratch_types=[
            pltpu.VMEM((ch_per_w, CHUNK), jnp.int32),
            pltpu.VMEM((ch_per_w, CHUNK), jnp.int32),
        ] + [pltpu.VMEM((CHUNK, DA), jnp.float32)] * NBUF + [
            pltpu.VMEM((zrows, DA), jnp.float32),
            pltpu.VMEM_SHARED((n_acc, DA), jnp.float32),
        ] + [pltpu.SemaphoreType.DMA] * NBUF,
    )
    def k(z_hbm, src_hbm, dst_hbm, out_hbm, src_v, dst_v,
          rows0, rows1, rows2, rows3, zero_v, acc_s, sem0, sem1, sem2, sem3):
        rows = (rows0, rows1, rows2, rows3)
        sems = (sem0, sem1, sem2, sem3)
        cid = lax.axis_index("c")
        sid = lax.axis_index("s")
        wid = sid * NC + cid

        def zrow(r, _):
            for cc in range(DA // 16):
                zero_v[r, pl.ds(cc * 16, 16)] = jnp.zeros((16,), jnp.float32)
            return 0

        lax.fori_loop(0, zrows, zrow, 0)
        pltpu.sync_copy(zero_v, acc_s.at[pl.ds(sid * zrows, zrows)])

        pltpu.sync_copy(src_hbm.at[wid], src_v)
        pltpu.sync_copy(dst_hbm.at[wid], dst_v)
        plsc.subcore_barrier()

        last = ch_per_w - 1
        for b in range(NBUF - 1):
            pltpu.async_copy(z_hbm.at[src_v.at[b]], rows[b], sems[b])

        def body(jj, _):
            for b in range(NBUF):
                j = jj * NBUF + b
                jn = jnp.minimum(j + NBUF - 1, last)
                bn = (b + NBUF - 1) % NBUF
                pltpu.async_copy(z_hbm.at[src_v.at[jn]], rows[bn], sems[bn])
                pltpu.make_async_copy(
                    z_hbm.at[src_v.at[j]], rows[b], sems[b]).wait()
                pltpu.sync_copy(rows[b], acc_s.at[dst_v.at[j]], add=True)
            return 0

        lax.fori_loop(0, ch_per_w // NBUF, body, 0)
        for b in range(NBUF - 1):
            bb = b % NBUF
            pltpu.make_async_copy(
                z_hbm.at[src_v.at[last]], rows[bb], sems[bb]).wait()
        plsc.subcore_barrier()
        pltpu.sync_copy(acc_s.at[pl.ds(sid * zrows, zrows)],
                        out_hbm.at[cid, pl.ds(sid * zrows, zrows)])

    return k(zaug, src3d, dst3d)


def _tc_layer1(h, wl1p, wr1p, b1p, dst2d):
    nblk = 8
    rows = N1 // nblk

    def body(h_ref, wl_ref, wr_ref, b_ref, d_ref, z_ref, hr_ref, dr_ref):
        i = pl.program_id(0)
        hb = h_ref[...]
        z = jnp.dot(hb, wl_ref[...], preferred_element_type=jnp.float32)
        col = lax.broadcasted_iota(jnp.int32, (rows, DA), 1)
        z_ref[...] = jnp.where((col >= D_HID) & (col < ONES_HI), 1.0, z)
        d = d_ref[...]
        dr_ref[...] = jnp.where(d < N2, d, N2 + (d & 63))

        @pl.when(i == 0)
        def _():
            hr_ref[...] = (
                jnp.dot(hb[:N2], wr_ref[...], preferred_element_type=jnp.float32)
                + b_ref[...])

    return pl.pallas_call(
        body,
        grid=(nblk,),
        in_specs=[
            pl.BlockSpec((rows, D_IN), lambda i: (i, 0)),
            pl.BlockSpec((D_IN, DA), lambda i: (0, 0)),
            pl.BlockSpec((D_IN, DA), lambda i: (0, 0)),
            pl.BlockSpec((1, DA), lambda i: (0, 0)),
            pl.BlockSpec((E1 // nblk // CHUNK, CHUNK), lambda i: (i, 0)),
        ],
        out_specs=[
            pl.BlockSpec((rows, DA), lambda i: (i, 0)),
            pl.BlockSpec((N2, DA), lambda i: (0, 0)),
            pl.BlockSpec((E1 // nblk // CHUNK, CHUNK), lambda i: (i, 0)),
        ],
        out_shape=[
            jax.ShapeDtypeStruct((N1, DA), jnp.float32),
            jax.ShapeDtypeStruct((N2, DA), jnp.float32),
            jax.ShapeDtypeStruct((E1 // CHUNK, CHUNK), jnp.int32),
        ],
    )(h, wl1p, wr1p, b1p, dst2d)


def _tc_relu_mean(parts, hr):
    def body(p_ref, hr_ref, out_ref):
        s = p_ref[0] + p_ref[1]
        deg = s[:, D_HID:D_HID + 1]
        meanf = s / jnp.maximum(deg, 1.0)
        pre = jnp.maximum(meanf + hr_ref[...], 0.0)
        col = lax.broadcasted_iota(jnp.int32, (N2, DA), 1)
        out_ref[...] = jnp.where((col >= D_HID) & (col < ONES_HI), 1.0, pre)

    return pl.pallas_call(
        body,
        out_shape=jax.ShapeDtypeStruct((N2, DA), jnp.float32),
    )(parts, hr)


def _tc_out(parts2, h1aug, wl2p, wr2p, b2):
    def body(p_ref, h1_ref, wl_ref, wr_ref, b_ref, out_ref):
        s = p_ref[0] + p_ref[1]
        deg = s[:, D_HID:D_HID + 1]
        meanf = s / jnp.maximum(deg, 1.0)
        out_ref[...] = (
            jnp.dot(meanf, wl_ref[...], preferred_element_type=jnp.float32)
            + jnp.dot(h1_ref[...], wr_ref[...], preferred_element_type=jnp.float32)
            + b_ref[...])

    return pl.pallas_call(
        body,
        out_shape=jax.ShapeDtypeStruct((N2, D_OUT), jnp.float32),
    )(parts2, h1aug, wl2p, wr2p, b2)


def kernel(x, n_id, edge_index1, edge_index2, Wl1, Wr1, b1, Wl2, Wr2, b2):
    idx = n_id[:N1].astype(jnp.int32).reshape(NW, N1 // NW // 64, 64)
    src1 = edge_index1[0].astype(jnp.int32).reshape(NW, E1 // CHUNK // NW, CHUNK)
    dst1f = edge_index1[1].astype(jnp.int32).reshape(E1 // CHUNK, CHUNK)
    src2 = edge_index2[0].astype(jnp.int32).reshape(NW, E2 // CHUNK // NW, CHUNK)
    dst2 = edge_index2[1].astype(jnp.int32).reshape(NW, E2 // CHUNK // NW, CHUNK)

    h = _gather_rows(x, idx, N1, D_IN)
    wl1p = jnp.pad(Wl1, ((0, 0), (0, DA - D_HID)))
    wr1p = jnp.pad(Wr1, ((0, 0), (0, DA - D_HID)))
    b1p = jnp.pad(b1, (0, DA - D_HID)).reshape(1, DA)
    wl2p = jnp.pad(Wl2, ((0, DA - D_HID), (0, 0)))
    wr2p = jnp.pad(Wr2, ((0, DA - D_HID), (0, 0)))
    zaug, hr, dst1r = _tc_layer1(h, wl1p, wr1p, b1p, dst1f)
    dst1 = dst1r.reshape(NW, E1 // CHUNK // NW, CHUNK)
    parts1 = _seg_accum(zaug, src1, dst1, redirect=True)
    h1aug = _tc_relu_mean(parts1, hr)
    parts2 = _seg_accum(h1aug, src2, dst2, redirect=False)
    return _tc_out(parts2, h1aug, wl2p, wr2p, b2.reshape(1, D_OUT))


# speedup vs baseline: 1.0341x; 1.0341x over previous
import functools

import jax
import jax.numpy as jnp
from jax import lax
from jax.experimental import pallas as pl
from jax.experimental.pallas import tpu as pltpu
from jax.experimental.pallas import tpu_sc as plsc

N0, N1, N2 = 40960, 10240, 1024
D_IN, D_HID, D_OUT = 256, 32, 256
E1, E2 = 163840, 16384
NC, NS = 2, 16
NW = NC * NS
DA = 128
ONES_HI = D_HID + 16
CHUNK = 128


def _sc_mesh():
    return plsc.VectorSubcoreMesh(core_axis_name="c", subcore_axis_name="s")


def _gather_rows(table, idx3d, n_rows, d):
    per_w = n_rows // NW
    n_sub = per_w // 64

    assert n_sub % 2 == 1

    @functools.partial(
        pl.kernel,
        mesh=_sc_mesh(),
        out_type=jax.ShapeDtypeStruct((n_rows, d), jnp.float32),
        scratch_types=[
            pltpu.VMEM((n_sub, 64), jnp.int32),
            pltpu.VMEM((64, d), jnp.float32),
            pltpu.VMEM((64, d), jnp.float32),
            pltpu.SemaphoreType.DMA,
            pltpu.SemaphoreType.DMA,
        ],
    )
    def k(table_hbm, idx_hbm, out_hbm, idx_v, rows0, rows1, sem0, sem1):
        cid = lax.axis_index("c")
        sid = lax.axis_index("s")
        wid = sid * NC + cid
        base = wid * per_w
        pltpu.sync_copy(idx_hbm.at[wid], idx_v)
        last = n_sub - 1
        pltpu.async_copy(table_hbm.at[idx_v.at[0]], rows0, sem0)

        def body(jj, _):
            j1 = jj * 2 + 1
            j0n = jnp.minimum(j1 + 1, last)
            pltpu.async_copy(table_hbm.at[idx_v.at[j1]], rows1, sem1)
            pltpu.make_async_copy(
                table_hbm.at[idx_v.at[j1 - 1]], rows0, sem0).wait()
            pltpu.sync_copy(rows0, out_hbm.at[pl.ds(base + (j1 - 1) * 64, 64)])
            pltpu.async_copy(table_hbm.at[idx_v.at[j0n]], rows0, sem0)
            pltpu.make_async_copy(
                table_hbm.at[idx_v.at[j1]], rows1, sem1).wait()
            pltpu.sync_copy(rows1, out_hbm.at[pl.ds(base + j1 * 64, 64)])
            return 0

        lax.fori_loop(0, n_sub // 2, body, 0)
        pltpu.make_async_copy(table_hbm.at[idx_v.at[last]], rows0, sem0).wait()
        pltpu.sync_copy(rows0, out_hbm.at[pl.ds(base + last * 64, 64)])

    return k(table, idx3d)


def _seg_accum(zaug, src3d, dst3d, redirect):
    n_acc = (N2 + 64) if redirect else N2
    ch_per_w = src3d.shape[1]
    zrows = N2 // NS
    assert ch_per_w % 4 == 0
    NBUF = 4

    @functools.partial(
        pl.kernel,
        mesh=_sc_mesh(),
        out_type=jax.ShapeDtypeStruct((NC, N2, DA), jnp.float32),
        scratch_types=[
            pltpu.VMEM((ch_per_w, CHUNK), jnp.int32),
            pltpu.VMEM((ch_per_w, CHUNK), jnp.int32),
        ] + [pltpu.VMEM((CHUNK, DA), jnp.float32)] * NBUF + [
            pltpu.VMEM((zrows, DA), jnp.float32),
            pltpu.VMEM_SHARED((n_acc, DA), jnp.float32),
        ] + [pltpu.SemaphoreType.DMA] * NBUF,
    )
    def k(z_hbm, src_hbm, dst_hbm, out_hbm, src_v, dst_v,
          rows0, rows1, rows2, rows3, zero_v, acc_s, sem0, sem1, sem2, sem3):
        rows = (rows0, rows1, rows2, rows3)
        sems = (sem0, sem1, sem2, sem3)
        cid = lax.axis_index("c")
        sid = lax.axis_index("s")
        wid = sid * NC + cid

        def zrow(r, _):
            for cc in range(DA // 16):
                zero_v[r, pl.ds(cc * 16, 16)] = jnp.zeros((16,), jnp.float32)
            return 0

        lax.fori_loop(0, zrows, zrow, 0)
        pltpu.sync_copy(zero_v, acc_s.at[pl.ds(sid * zrows, zrows)])

        pltpu.sync_copy(src_hbm.at[wid], src_v)
        pltpu.sync_copy(dst_hbm.at[wid], dst_v)
        plsc.subcore_barrier()

        last = ch_per_w - 1
        for b in range(NBUF - 1):
            pltpu.async_copy(z_hbm.at[src_v.at[b]], rows[b], sems[b])

        def body(jj, _):
            for b in range(NBUF):
                j = jj * NBUF + b
                jn = j + NBUF - 1
                bn = (b + NBUF - 1) % NBUF

                @pl.when(jn <= last)
                def _():
                    pltpu.async_copy(z_hbm.at[src_v.at[jn]], rows[bn], sems[bn])

                pltpu.make_async_copy(
                    z_hbm.at[src_v.at[j]], rows[b], sems[b]).wait()
                pltpu.sync_copy(rows[b], acc_s.at[dst_v.at[j]], add=True)
            return 0

        lax.fori_loop(0, ch_per_w // NBUF, body, 0)
        plsc.subcore_barrier()
        pltpu.sync_copy(acc_s.at[pl.ds(sid * zrows, zrows)],
                        out_hbm.at[cid, pl.ds(sid * zrows, zrows)])

    return k(zaug, src3d, dst3d)


def _tc_layer1(h, wl1p, wr1p, b1p, dst2d):
    nblk = 8
    rows = N1 // nblk

    def body(h_ref, wl_ref, wr_ref, b_ref, d_ref, z_ref, hr_ref, dr_ref):
        i = pl.program_id(0)
        hb = h_ref[...]
        z = jnp.dot(hb, wl_ref[...], preferred_element_type=jnp.float32)
        col = lax.broadcasted_iota(jnp.int32, (rows, DA), 1)
        z_ref[...] = jnp.where((col >= D_HID) & (col < ONES_HI), 1.0, z)
        d = d_ref[...]
        dr_ref[...] = jnp.where(d < N2, d, N2 + (d & 63))

        @pl.when(i == 0)
        def _():
            hr_ref[...] = (
                jnp.dot(hb[:N2], wr_ref[...], preferred_element_type=jnp.float32)
                + b_ref[...])

    return pl.pallas_call(
        body,
        grid=(nblk,),
        in_specs=[
            pl.BlockSpec((rows, D_IN), lambda i: (i, 0)),
            pl.BlockSpec((D_IN, DA), lambda i: (0, 0)),
            pl.BlockSpec((D_IN, DA), lambda i: (0, 0)),
            pl.BlockSpec((1, DA), lambda i: (0, 0)),
            pl.BlockSpec((E1 // nblk // CHUNK, CHUNK), lambda i: (i, 0)),
        ],
        out_specs=[
            pl.BlockSpec((rows, DA), lambda i: (i, 0)),
            pl.BlockSpec((N2, DA), lambda i: (0, 0)),
            pl.BlockSpec((E1 // nblk // CHUNK, CHUNK), lambda i: (i, 0)),
        ],
        out_shape=[
            jax.ShapeDtypeStruct((N1, DA), jnp.float32),
            jax.ShapeDtypeStruct((N2, DA), jnp.float32),
            jax.ShapeDtypeStruct((E1 // CHUNK, CHUNK), jnp.int32),
        ],
    )(h, wl1p, wr1p, b1p, dst2d)


def _tc_relu_mean(parts, hr):
    def body(p_ref, hr_ref, out_ref):
        s = p_ref[0] + p_ref[1]
        deg = s[:, D_HID:D_HID + 1]
        meanf = s / jnp.maximum(deg, 1.0)
        pre = jnp.maximum(meanf + hr_ref[...], 0.0)
        col = lax.broadcasted_iota(jnp.int32, (N2, DA), 1)
        out_ref[...] = jnp.where((col >= D_HID) & (col < ONES_HI), 1.0, pre)

    return pl.pallas_call(
        body,
        out_shape=jax.ShapeDtypeStruct((N2, DA), jnp.float32),
    )(parts, hr)


def _tc_out(parts2, h1aug, wl2p, wr2p, b2):
    def body(p_ref, h1_ref, wl_ref, wr_ref, b_ref, out_ref):
        s = p_ref[0] + p_ref[1]
        deg = s[:, D_HID:D_HID + 1]
        meanf = s / jnp.maximum(deg, 1.0)
        out_ref[...] = (
            jnp.dot(meanf, wl_ref[...], preferred_element_type=jnp.float32)
            + jnp.dot(h1_ref[...], wr_ref[...], preferred_element_type=jnp.float32)
            + b_ref[...])

    return pl.pallas_call(
        body,
        out_shape=jax.ShapeDtypeStruct((N2, D_OUT), jnp.float32),
    )(parts2, h1aug, wl2p, wr2p, b2)


def kernel(x, n_id, edge_index1, edge_index2, Wl1, Wr1, b1, Wl2, Wr2, b2):
    idx = n_id[:N1].astype(jnp.int32).reshape(NW, N1 // NW // 64, 64)
    src1 = edge_index1[0].astype(jnp.int32).reshape(NW, E1 // CHUNK // NW, CHUNK)
    dst1f = edge_index1[1].astype(jnp.int32).reshape(E1 // CHUNK, CHUNK)
    src2 = edge_index2[0].astype(jnp.int32).reshape(NW, E2 // CHUNK // NW, CHUNK)
    dst2 = edge_index2[1].astype(jnp.int32).reshape(NW, E2 // CHUNK // NW, CHUNK)

    h = _gather_rows(x, idx, N1, D_IN)
    wl1p = jnp.pad(Wl1, ((0, 0), (0, DA - D_HID)))
    wr1p = jnp.pad(Wr1, ((0, 0), (0, DA - D_HID)))
    b1p = jnp.pad(b1, (0, DA - D_HID)).reshape(1, DA)
    wl2p = jnp.pad(Wl2, ((0, DA - D_HID), (0, 0)))
    wr2p = jnp.pad(Wr2, ((0, DA - D_HID), (0, 0)))
    zaug, hr, dst1r = _tc_layer1(h, wl1p, wr1p, b1p, dst1f)
    dst1 = dst1r.reshape(NW, E1 // CHUNK // NW, CHUNK)
    parts1 = _seg_accum(zaug, src1, dst1, redirect=True)
    h1aug = _tc_relu_mean(parts1, hr)
    parts2 = _seg_accum(h1aug, src2, dst2, redirect=False)
    return _tc_out(parts2, h1aug, wl2p, wr2p, b2.reshape(1, D_OUT))
